# baseline (speedup 1.0000x reference)
import functools

import jax
import jax.numpy as jnp
from jax import lax
from jax.experimental import pallas as pl
from jax.experimental.pallas import tpu as pltpu
from jax.experimental.pallas import tpu_sc as plsc

_B = 64
_T = 8192
_C = 16
_N = _B * _T
_NW = 32
_PER_W = _N // _NW
_CH = 2048
_SL = _CH + 16
_NCHUNK = _PER_W // _CH


def _locnet_body(x_ref, k_ref, bias_ref, r0_ref, r1_ref, w0_ref, w1_ref, *, t_len):
    bidx = pl.program_id(0)
    A = x_ref[0]
    K = k_ref[...]
    raw = jax.lax.dot(A, K, precision=jax.lax.Precision.HIGHEST,
                      preferred_element_type=jnp.float32)
    d = jnp.tanh(raw + bias_ref[0, 0])

    r_iota = jax.lax.broadcasted_iota(jnp.int32, (64, 128), 0)
    l_iota = jax.lax.broadcasted_iota(jnp.int32, (64, 128), 1)
    t_i = r_iota * 128 + l_iota
    t = t_i.astype(jnp.float32)

    x = t + d
    x0 = jnp.floor(x)
    x1 = x0 + 1.0
    fmax = float(t_len - 1)
    x0c = jnp.clip(x0, 0.0, fmax)
    x1c = jnp.clip(x1, 0.0, fmax)
    w0_ref[0] = x1c - x
    w1_ref[0] = x - x0c

    i_flat = bidx * t_len + t_i
    chunk_start = (i_flat // _CH) * _CH
    slice_start = jnp.clip(chunk_start - 8, 0, _N - _SL)
    base = bidx * t_len
    r0_ref[0] = ((base + x0c.astype(jnp.int32)) - slice_start) * _C
    r1_ref[0] = ((base + x1c.astype(jnp.int32)) - slice_start) * _C


def _sc_interp_body(table_ref, r0_hbm, r1_hbm, w0_hbm, w1_hbm, out_hbm,
                    table_v, r0_v, r1_v, w0_v, w1_v, out_v):
    wid = lax.axis_index("s") * 2 + lax.axis_index("c")

    def chunk_body(j, carry):
        cs = pl.multiple_of(wid * _PER_W + j * _CH, _CH)
        ss = pl.multiple_of(jnp.clip(cs - 8, 0, _N - _SL), 8)
        pltpu.sync_copy(table_ref.at[pl.ds(ss * _C, _SL * _C)], table_v)
        pltpu.sync_copy(r0_hbm.at[pl.ds(cs, _CH)], r0_v)
        pltpu.sync_copy(r1_hbm.at[pl.ds(cs, _CH)], r1_v)
        pltpu.sync_copy(w0_hbm.at[pl.ds(cs, _CH)], w0_v)
        pltpu.sync_copy(w1_hbm.at[pl.ds(cs, _CH)], w1_v)

        def blk(i, carry2):
            base = i * 16
            addr0 = r0_v[pl.ds(base, 16)]
            addr1 = r1_v[pl.ds(base, 16)]
            w0v = w0_v[pl.ds(base, 16)]
            w1v = w1_v[pl.ds(base, 16)]
            oaddr = (lax.iota(jnp.int32, 16) + base) * _C
            for c in range(_C):
                v0 = plsc.load_gather(table_v, [addr0 + c])
                v1 = plsc.load_gather(table_v, [addr1 + c])
                plsc.store_scatter(out_v, [oaddr + c], w0v * v0 + w1v * v1)
            return carry2

        lax.fori_loop(0, _CH // 16, blk, 0)
        pltpu.sync_copy(out_v, out_hbm.at[pl.ds(cs * _C, _CH * _C)])
        return carry

    lax.fori_loop(0, _NCHUNK, chunk_body, 0)


def kernel(X, Wc, b):
    B, T, C = X.shape

    k_idx = jnp.arange(2048)
    tpos = 8 * (k_idx // 128) + (k_idx % 128) // 16
    K = jnp.where(tpos[:, None] == jnp.arange(128)[None, :],
                  Wc[:, 0][k_idx % 16][:, None], 0.0).astype(jnp.float32)
    bias = b.reshape(1, 1).astype(jnp.float32)
    Xc = X.reshape(B, T // 128, 2048)

    grid_spec = pl.GridSpec(
        grid=(B,),
        in_specs=[
            pl.BlockSpec((1, T // 128, 2048), lambda i: (i, 0, 0)),
            pl.BlockSpec((2048, 128), lambda i: (0, 0)),
            pl.BlockSpec((1, 1), lambda i: (0, 0)),
        ],
        out_specs=[pl.BlockSpec((1, 64, 128), lambda i: (i, 0, 0))] * 4,
    )
    shp = jax.ShapeDtypeStruct((B, 64, 128), jnp.float32)
    shpi = jax.ShapeDtypeStruct((B, 64, 128), jnp.int32)
    r0, r1, w0, w1 = pl.pallas_call(
        functools.partial(_locnet_body, t_len=T),
        grid_spec=grid_spec,
        out_shape=[shpi, shpi, shp, shp],
    )(Xc, K, bias)

    flat = X.reshape(_N * _C)
    mesh = plsc.VectorSubcoreMesh(core_axis_name="c", subcore_axis_name="s")
    sc = pl.kernel(
        _sc_interp_body,
        mesh=mesh,
        compiler_params=pltpu.CompilerParams(needs_layout_passes=False),
        out_type=jax.ShapeDtypeStruct((_N * _C,), jnp.float32),
        scratch_types=[
            pltpu.VMEM((_SL * _C,), jnp.float32),
            pltpu.VMEM((_CH,), jnp.int32),
            pltpu.VMEM((_CH,), jnp.int32),
            pltpu.VMEM((_CH,), jnp.float32),
            pltpu.VMEM((_CH,), jnp.float32),
            pltpu.VMEM((_CH * _C,), jnp.float32),
        ],
    )
    out = sc(flat, r0.reshape(_N), r1.reshape(_N), w0.reshape(_N), w1.reshape(_N))
    return out.reshape(B, _T, _C)


# speedup vs baseline: 3.7647x; 3.7647x over previous
import functools

import jax
import jax.numpy as jnp
from jax import lax
from jax.experimental import pallas as pl
from jax.experimental.pallas import tpu as pltpu
from jax.experimental.pallas import tpu_sc as plsc

_B = 64
_T = 8192
_C = 16
_NW = 32
_CH = 2048
_SLW = _CH + 256
_CHUNKS_PB = _T // _CH
_TASKS = _B * _CHUNKS_PB
_TASKS_PW = _TASKS // _NW


def _locnet_body(x_ref, wc_ref, bias_ref, r0_ref, r1_ref, w0_ref, w1_ref, *, nb):
    A = x_ref[...]
    acc = A[:, 0, :] * wc_ref[0, 0]
    for c in range(1, _C):
        acc = acc + A[:, c, :] * wc_ref[c, 0]
    d = jnp.tanh(acc + bias_ref[0, 0])

    t_i = lax.broadcasted_iota(jnp.int32, (nb, _T), 1)
    t = t_i.astype(jnp.float32)
    x = t + d
    x0 = jnp.floor(x)
    x1 = x0 + 1.0
    fmax = float(_T - 1)
    x0c = jnp.clip(x0, 0.0, fmax)
    x1c = jnp.clip(x1, 0.0, fmax)
    w0_ref[...] = x1c - x
    w1_ref[...] = x - x0c

    chunk_start = (t_i // _CH) * _CH
    slice_start = jnp.clip(chunk_start - 128, 0, _T - _SLW)
    r0_ref[...] = x0c.astype(jnp.int32) - slice_start
    r1_ref[...] = x1c.astype(jnp.int32) - slice_start


def _sc_interp_body(xt_ref, r0_hbm, r1_hbm, w0_hbm, w1_hbm, out_hbm,
                    sig_v, r0_v, r1_v, w0_v, w1_v, out_v):
    wid = lax.axis_index("s") * 2 + lax.axis_index("c")

    def task_body(k, carry):
        task = wid * _TASKS_PW + k
        b = task // _CHUNKS_PB
        cs = pl.multiple_of((task % _CHUNKS_PB) * _CH, _CH)
        ss = pl.multiple_of(jnp.clip(cs - 128, 0, _T - _SLW), 128)
        pltpu.sync_copy(xt_ref.at[b, :, pl.ds(ss, _SLW)], sig_v)
        pltpu.sync_copy(r0_hbm.at[b, pl.ds(cs, _CH)], r0_v)
        pltpu.sync_copy(r1_hbm.at[b, pl.ds(cs, _CH)], r1_v)
        pltpu.sync_copy(w0_hbm.at[b, pl.ds(cs, _CH)], w0_v)
        pltpu.sync_copy(w1_hbm.at[b, pl.ds(cs, _CH)], w1_v)

        def blk(i, carry2):
            t0 = i * 16
            pos0 = r0_v[pl.ds(t0, 16)]
            pos1 = r1_v[pl.ds(t0, 16)]
            w0v = w0_v[pl.ds(t0, 16)]
            w1v = w1_v[pl.ds(t0, 16)]
            for c in range(_C):
                cvec = jnp.full((16,), c, jnp.int32)
                v0 = plsc.load_gather(sig_v, [cvec, pos0])
                v1 = plsc.load_gather(sig_v, [cvec, pos1])
                out_v[c, pl.ds(t0, 16)] = w0v * v0 + w1v * v1
            return carry2

        lax.fori_loop(0, _CH // 16, blk, 0)
        pltpu.sync_copy(out_v, out_hbm.at[b, :, pl.ds(cs, _CH)])
        return carry

    lax.fori_loop(0, _TASKS_PW, task_body, 0)


def kernel(X, Wc, b):
    B, T, C = X.shape
    XT = X.transpose(0, 2, 1)
    nb = 8

    wc = Wc.astype(jnp.float32)
    bias = b.reshape(1, 1).astype(jnp.float32)

    grid_spec = pl.GridSpec(
        grid=(B // nb,),
        in_specs=[
            pl.BlockSpec((nb, C, T), lambda i: (i, 0, 0)),
            pl.BlockSpec(memory_space=pltpu.SMEM),
            pl.BlockSpec(memory_space=pltpu.SMEM),
        ],
        out_specs=[pl.BlockSpec((nb, T), lambda i: (i, 0))] * 4,
    )
    shp = jax.ShapeDtypeStruct((B, T), jnp.float32)
    shpi = jax.ShapeDtypeStruct((B, T), jnp.int32)
    r0, r1, w0, w1 = pl.pallas_call(
        functools.partial(_locnet_body, nb=nb),
        grid_spec=grid_spec,
        out_shape=[shpi, shpi, shp, shp],
    )(XT, wc, bias)

    mesh = plsc.VectorSubcoreMesh(core_axis_name="c", subcore_axis_name="s")
    sc = pl.kernel(
        _sc_interp_body,
        mesh=mesh,
        compiler_params=pltpu.CompilerParams(needs_layout_passes=False),
        out_type=jax.ShapeDtypeStruct((B, C, T), jnp.float32),
        scratch_types=[
            pltpu.VMEM((_C, _SLW), jnp.float32),
            pltpu.VMEM((_CH,), jnp.int32),
            pltpu.VMEM((_CH,), jnp.int32),
            pltpu.VMEM((_CH,), jnp.float32),
            pltpu.VMEM((_CH,), jnp.float32),
            pltpu.VMEM((_C, _CH), jnp.float32),
        ],
    )
    out = sc(XT, r0, r1, w0, w1)
    return out.transpose(0, 2, 1)
